# baseline (speedup 1.0000x reference)
import jax
import jax.numpy as jnp
from jax.experimental import pallas as pl

NUM_NODES = 100000
NUM_GRAPHS = 1024
HID_DIM = 128

_BLK = 2000


def _nw_body(nf_ref, w_ref, b_ref, nw_ref):
    blk = nf_ref[...]
    w = w_ref[...]
    nw_ref[...] = jnp.sum(blk * w, axis=1, keepdims=True) + b_ref[0, 0]


def kernel(nf, segment_ids, W, b):
    seg = segment_ids.astype(jnp.int32)
    b2 = b.reshape(1, 1)
    nw = pl.pallas_call(
        _nw_body,
        grid=(NUM_NODES // _BLK,),
        in_specs=[
            pl.BlockSpec((_BLK, HID_DIM), lambda i: (i, 0)),
            pl.BlockSpec((1, HID_DIM), lambda i: (0, 0)),
            pl.BlockSpec((1, 1), lambda i: (0, 0), memory_space=pltpu_smem()),
        ],
        out_specs=pl.BlockSpec((_BLK, 1), lambda i: (i, 0)),
        out_shape=jax.ShapeDtypeStruct((NUM_NODES, 1), jnp.float32),
    )(nf, W, b2)

    seg_max = jax.ops.segment_max(nw, seg, num_segments=NUM_GRAPHS)
    seg_max = jnp.where(jnp.isfinite(seg_max), seg_max, 0.0)
    e = jnp.exp(nw - seg_max[seg])
    denom = jax.ops.segment_sum(e, seg, num_segments=NUM_GRAPHS)
    weights = e / denom[seg]
    weighted = jax.ops.segment_sum(weights * nf, seg, num_segments=NUM_GRAPHS)
    max_rd = jax.ops.segment_max(nf, seg, num_segments=NUM_GRAPHS)
    out = jnp.concatenate([weighted, max_rd], axis=1)
    return out, weights


def pltpu_smem():
    from jax.experimental.pallas import tpu as pltpu
    return pltpu.SMEM


# speedup vs baseline: 10.2903x; 10.2903x over previous
import dataclasses
import functools

import jax
import jax.numpy as jnp
from jax import lax
from jax.experimental import pallas as pl
from jax.experimental.pallas import tpu as pltpu
from jax.experimental.pallas import tpu_sc as plsc

N = 100000
G = 1024
D = 128
L = 16
NW = 32
REC_W = 272
NEG_INF = float("-inf")

_mesh = plsc.VectorSubcoreMesh(core_axis_name="c", subcore_axis_name="s")

_cparams = pltpu.CompilerParams()
if "needs_layout_passes" in pltpu.CompilerParams.__dataclass_fields__:
    _cparams = dataclasses.replace(_cparams, needs_layout_passes=False)


def _bs(x):
    return lax.broadcast_in_dim(jnp.float32(0) + x, (L,), ())


def _bs_i(x):
    return lax.broadcast_in_dim(jnp.int32(0) + x, (L,), ())


def _lane_iota():
    return lax.broadcasted_iota(jnp.int32, (L,), 0)


def _tile_id():
    return lax.axis_index("s") * 2 + lax.axis_index("c")


def _a_body(nf_hbm, ids_hbm, wb_hbm, nw_hbm, outa_hbm, msa_hbm, recs_hbm,
            rid_hbm, buf, idbuf, nwbuf, w_v, b_v, st_f, s_v, acc, fmx,
            stage, ms_stage, rec_v, rid_v, empty_row, nwv_s, st_i, sm_ids,
            in_sem, fl_sem, init_sem):
    wid = _tile_id()
    row0 = 3120 * wid + 16 * jnp.minimum(wid, 10)
    nc4 = jnp.where(wid < 10, 49, 48)

    pltpu.sync_copy(wb_hbm.at[pl.ds(0, 128)], w_v)
    pltpu.sync_copy(wb_hbm.at[pl.ds(128, 16)], b_v)
    b_s = b_v[...][0]

    pltpu.sync_copy(ids_hbm.at[pl.ds(row0, 16)], idbuf.at[0, pl.ds(0, 16)])
    last_row0 = row0 + jnp.where(wid < 10, 3136 - 16, 3120 - 16)
    pltpu.sync_copy(ids_hbm.at[pl.ds(last_row0, 16)],
                    idbuf.at[0, pl.ds(16, 16)])
    first_seg = idbuf[0, pl.ds(0, 16)][0]
    last_seg = idbuf[0, pl.ds(16, 16)][15]
    st_i[0] = first_seg
    st_i[1] = 0
    st_i[2] = first_seg
    st_f[0] = jnp.float32(NEG_INF)

    lane = _lane_iota()
    zeros = jnp.zeros((L,), jnp.float32)
    ninf = jnp.full((L,), NEG_INF, jnp.float32)

    for j in range(8):
        acc[pl.ds(16 * j, 16)] = zeros
        fmx[pl.ds(16 * j, 16)] = ninf
        empty_row[pl.ds(16 * j, 16)] = zeros
        empty_row[pl.ds(128 + 16 * j, 16)] = ninf
        for sl in range(2):
            rec_v[sl, pl.ds(16 * j, 16)] = zeros
            rec_v[sl, pl.ds(128 + 16 * j, 16)] = ninf
    s_v[...] = zeros
    ms_neutral = jnp.where(lane == 0, ninf, zeros)
    rec_v[0, pl.ds(256, 16)] = ms_neutral
    rec_v[1, pl.ds(256, 16)] = ms_neutral

    @pl.loop(first_seg + 1, last_seg)
    def _(g):
        pltpu.async_copy(empty_row, outa_hbm.at[g], init_sem)

    @pl.loop(first_seg + 1, last_seg)
    def _(g):
        pltpu.make_async_copy(empty_row, outa_hbm.at[g], init_sem).wait()

    def build_msvec():
        m = st_f[0]
        s_sc = jnp.max(s_v[...])
        return jnp.where(lane == 0, _bs(m),
                         jnp.where(lane == 1, _bs(s_sc), zeros))

    def write_record(islot):
        for j in range(8):
            rec_v[islot, pl.ds(16 * j, 16)] = acc[pl.ds(16 * j, 16)]
            rec_v[islot, pl.ds(128 + 16 * j, 16)] = fmx[pl.ds(16 * j, 16)]
        rec_v[islot, pl.ds(256, 16)] = build_msvec()

    def flush(cur):
        is_first = cur == st_i[2]

        @pl.when(is_first)
        def _():
            write_record(0)

        @pl.when(jnp.logical_not(is_first))
        def _():
            f = st_i[1]
            slot = lax.rem(f, 2)

            @pl.when(f >= 2)
            def _():
                pltpu.make_async_copy(stage.at[slot], outa_hbm.at[0],
                                      fl_sem.at[slot]).wait()
                pltpu.make_async_copy(ms_stage.at[slot], msa_hbm.at[0],
                                      fl_sem.at[slot]).wait()

            inv = 1.0 / s_v[...]
            for j in range(8):
                stage[slot, pl.ds(16 * j, 16)] = acc[pl.ds(16 * j, 16)] * inv
                stage[slot, pl.ds(128 + 16 * j, 16)] = fmx[pl.ds(16 * j, 16)]
            ms_stage[slot, pl.ds(0, 16)] = build_msvec()
            pltpu.async_copy(stage.at[slot], outa_hbm.at[cur],
                             fl_sem.at[slot])
            pltpu.async_copy(ms_stage.at[slot], msa_hbm.at[cur],
                             fl_sem.at[slot])
            st_i[1] = f + 1

    def reset_state(sid):
        for j in range(8):
            acc[pl.ds(16 * j, 16)] = zeros
            fmx[pl.ds(16 * j, 16)] = ninf
        s_v[...] = zeros
        st_f[0] = jnp.float32(NEG_INF)
        st_i[0] = sid

    def row_score(slot, rr, wregs):
        vs = [buf[slot, rr, pl.ds(16 * j, 16)] for j in range(8)]
        a = vs[0] * wregs[0]
        for j in range(1, 8):
            a = a + vs[j] * wregs[j]
        return jnp.sum(a) + b_s, vs

    def process_group(slot, goff, vec_idx):
        wregs = [w_v[pl.ds(16 * j, 16)] for j in range(8)]
        gv = idbuf[slot, pl.ds(goff, 16)]
        g0s = gv[0]
        cur0 = st_i[0]
        allsame = jnp.min((gv == _bs_i(g0s)).astype(jnp.int32)) == 1
        single = jnp.logical_and(allsame, g0s == cur0)

        @pl.when(single)
        def _():
            nwv = zeros
            rows = []
            for r in range(16):
                nw_s, vs = row_score(slot, goff + r, wregs)
                nwv = jnp.where(lane == r, _bs(nw_s), nwv)
            nwbuf[pl.ds(16 * vec_idx, 16)] = nwv
            gm = jnp.max(nwv)
            m_old = st_f[0]
            new_m = jnp.maximum(m_old, gm)
            sc_v = jnp.exp(_bs(m_old - new_m))
            e_vec = jnp.exp(nwv - _bs(new_m))
            s_v[...] = s_v[...] * sc_v + _bs(jnp.sum(e_vec))
            st_f[0] = new_m
            e_bs = [_bs(e_vec[r]) for r in range(16)]
            for j in range(8):
                sl = pl.ds(16 * j, 16)
                a_j = acc[sl] * sc_v
                f_j = fmx[sl]
                for r in range(16):
                    v = buf[slot, goff + r, sl]
                    a_j = a_j + e_bs[r] * v
                    f_j = jnp.maximum(f_j, v)
                acc[sl] = a_j
                fmx[sl] = f_j

        @pl.when(jnp.logical_not(single))
        def _():
            for r in range(16):
                sm_ids[r] = gv[r]
            nwv_s[...] = zeros

            @pl.loop(0, 16)
            def _(r):
                rr = goff + r
                sid = sm_ids[r]
                cur = st_i[0]

                @pl.when(sid != cur)
                def _():
                    flush(cur)
                    reset_state(sid)

                nw_s, vs = row_score(slot, rr, wregs)
                nwv_s[...] = jnp.where(lane == r, _bs(nw_s), nwv_s[...])

                m_old = st_f[0]
                new_m = jnp.maximum(m_old, nw_s)
                sc_v = jnp.exp(_bs(m_old - new_m))
                e_v = jnp.exp(_bs(nw_s - new_m))
                s_v[...] = s_v[...] * sc_v + e_v
                st_f[0] = new_m
                for j in range(8):
                    sl = pl.ds(16 * j, 16)
                    acc[sl] = acc[sl] * sc_v + e_v * vs[j]
                    fmx[sl] = jnp.maximum(fmx[sl], vs[j])

            nwbuf[pl.ds(16 * vec_idx, 16)] = nwv_s[...]

    def fire_chunk(ci, slot):
        pltpu.async_copy(nf_hbm.at[pl.ds(row0 + 64 * ci, 64), :],
                         buf.at[slot], in_sem.at[slot])
        pltpu.async_copy(ids_hbm.at[pl.ds(row0 + 64 * ci, 64)],
                         idbuf.at[slot], in_sem.at[slot])

    def wait_chunk(slot):
        pltpu.make_async_copy(nf_hbm.at[pl.ds(0, 64), :], buf.at[slot],
                              in_sem.at[slot]).wait()
        pltpu.make_async_copy(ids_hbm.at[pl.ds(0, 64)], idbuf.at[slot],
                              in_sem.at[slot]).wait()

    fire_chunk(0, 0)
    fire_chunk(1, 1)

    @pl.loop(0, nc4)
    def _(ci):
        slot = lax.rem(ci, 2)
        wait_chunk(slot)

        @pl.loop(0, 4)
        def _(g):
            process_group(slot, 16 * g, 4 * ci + g)

        @pl.when(ci + 2 < nc4)
        def _():
            fire_chunk(ci + 2, slot)

    @pl.when(wid >= 10)
    def _():
        pltpu.sync_copy(nf_hbm.at[pl.ds(row0 + 3072, 48), :],
                        buf.at[0, pl.ds(0, 48), :])
        pltpu.sync_copy(ids_hbm.at[pl.ds(row0 + 3072, 48)],
                        idbuf.at[0, pl.ds(0, 48)])

        @pl.loop(0, 3)
        def _(t):
            process_group(0, 16 * t, 192 + t)

    cur = st_i[0]
    is_first = cur == st_i[2]
    islot = jnp.where(is_first, 0, 1)
    write_record(islot)
    rid_v[...] = jnp.where(lane == 0, _bs_i(st_i[2]),
                           jnp.where(lane == 1, _bs_i(cur),
                                     jnp.zeros((L,), jnp.int32)))

    f = st_i[1]

    @pl.when(f >= 1)
    def _():
        slot = lax.rem(f - 1, 2)
        pltpu.make_async_copy(stage.at[slot], outa_hbm.at[0],
                              fl_sem.at[slot]).wait()
        pltpu.make_async_copy(ms_stage.at[slot], msa_hbm.at[0],
                              fl_sem.at[slot]).wait()

    @pl.when(f >= 2)
    def _():
        slot = lax.rem(f, 2)
        pltpu.make_async_copy(stage.at[slot], outa_hbm.at[0],
                              fl_sem.at[slot]).wait()
        pltpu.make_async_copy(ms_stage.at[slot], msa_hbm.at[0],
                              fl_sem.at[slot]).wait()

    pltpu.sync_copy(rec_v, recs_hbm.at[wid])
    pltpu.sync_copy(rid_v, rid_hbm.at[wid])

    @pl.when(wid < 10)
    def _():
        pltpu.sync_copy(nwbuf.at[pl.ds(0, 3136)],
                        nw_hbm.at[pl.ds(row0, 3136)])

    @pl.when(wid >= 10)
    def _():
        pltpu.sync_copy(nwbuf.at[pl.ds(0, 3120)],
                        nw_hbm.at[pl.ds(row0, 3120)])


def _b_body(outa_hbm, msa_hbm, recs_hbm, rid_hbm, out_hbm, mf_hbm, df_hbm,
            rid_v, obuf, msbuf, rbuf, tacc, tmx, s_v, mstage, dstage,
            st_f, st_b):
    t = _tile_id()
    g0 = 32 * t
    lane = _lane_iota()
    zeros = jnp.zeros((L,), jnp.float32)
    ninf = jnp.full((L,), NEG_INF, jnp.float32)

    pltpu.sync_copy(rid_hbm, rid_v)
    pltpu.sync_copy(outa_hbm.at[pl.ds(g0, 32), :], obuf)
    pltpu.sync_copy(msa_hbm.at[pl.ds(g0, 32), :], msbuf)

    def set_lane(ref, pos, valv):
        base = 16 * lax.div(pos, 16)
        lpos = lax.rem(pos, 16)
        v = ref[pl.ds(base, 16)]
        ref[pl.ds(base, 16)] = jnp.where(lane == lpos, valv, v)

    @pl.loop(0, 32)
    def _(gl):
        g = g0 + gl

        st_b[0] = 0
        st_b[1] = 0

        @pl.loop(0, NW)
        def _(w):
            rv = rid_v[w, pl.ds(0, 16)]
            fw = rv[0]
            lw = rv[1]
            bh = jnp.logical_or(fw == g, lw == g).astype(jnp.int32)
            ih = jnp.logical_and(fw < g, g < lw).astype(jnp.int32)
            st_b[0] = st_b[0] | bh
            st_b[1] = st_b[1] | ih

        bhit = st_b[0] != 0
        ihit = st_b[1] != 0

        @pl.when(bhit)
        def _():
            for j in range(8):
                tacc[pl.ds(16 * j, 16)] = zeros
                tmx[pl.ds(16 * j, 16)] = ninf
            s_v[...] = zeros
            st_f[0] = jnp.float32(NEG_INF)

            @pl.loop(0, NW)
            def _(w):
                rv = rid_v[w, pl.ds(0, 16)]

                def merge():
                    msv = rbuf[pl.ds(256, 16)]
                    m_j = msv[0]
                    sg_j = msv[1]
                    m_old = st_f[0]
                    new_m = jnp.maximum(m_old, m_j)
                    sc_old = jnp.exp(_bs(m_old - new_m))
                    sc_j = jnp.exp(_bs(m_j - new_m))
                    s_v[...] = s_v[...] * sc_old + _bs(sg_j) * sc_j
                    st_f[0] = new_m
                    for j in range(8):
                        sli = pl.ds(16 * j, 16)
                        slo = pl.ds(128 + 16 * j, 16)
                        tacc[sli] = tacc[sli] * sc_old + rbuf[sli] * sc_j
                        tmx[sli] = jnp.maximum(tmx[sli], rbuf[slo])

                @pl.when(rv[0] == g)
                def _():
                    pltpu.sync_copy(recs_hbm.at[w, 0], rbuf)
                    merge()

                @pl.when(rv[1] == g)
                def _():
                    pltpu.sync_copy(recs_hbm.at[w, 1], rbuf)
                    merge()

            inv = 1.0 / s_v[...]
            for j in range(8):
                obuf[gl, pl.ds(16 * j, 16)] = tacc[pl.ds(16 * j, 16)] * inv
                obuf[gl, pl.ds(128 + 16 * j, 16)] = tmx[pl.ds(16 * j, 16)]
            set_lane(mstage, gl, _bs(st_f[0]))
            set_lane(dstage, gl, inv)

        @pl.when(jnp.logical_and(jnp.logical_not(bhit), ihit))
        def _():
            mrow = msbuf[gl, pl.ds(0, 16)]
            set_lane(mstage, gl, _bs(mrow[0]))
            set_lane(dstage, gl, 1.0 / _bs(mrow[1]))

        @pl.when(jnp.logical_and(jnp.logical_not(bhit),
                                 jnp.logical_not(ihit)))
        def _():
            for j in range(8):
                obuf[gl, pl.ds(16 * j, 16)] = zeros
                obuf[gl, pl.ds(128 + 16 * j, 16)] = ninf
            set_lane(mstage, gl, zeros)
            set_lane(dstage, gl, zeros)

    pltpu.sync_copy(obuf, out_hbm.at[pl.ds(g0, 32), :])
    pltpu.sync_copy(mstage, mf_hbm.at[pl.ds(g0, 32)])
    pltpu.sync_copy(dstage, df_hbm.at[pl.ds(g0, 32)])


def _c_body(nw_hbm, ids_hbm, mf_hbm, df_hbm, w_hbm, mv, dv, nwc, idc, wc):
    wid = _tile_id()
    row0 = 3120 * wid + 16 * jnp.minimum(wid, 10)
    nv = jnp.where(wid < 10, 196, 195)

    pltpu.sync_copy(mf_hbm, mv)
    pltpu.sync_copy(df_hbm, dv)

    @pl.when(wid < 10)
    def _():
        pltpu.sync_copy(nw_hbm.at[pl.ds(row0, 3136)], nwc.at[pl.ds(0, 3136)])
        pltpu.sync_copy(ids_hbm.at[pl.ds(row0, 3136)], idc.at[pl.ds(0, 3136)])

    @pl.when(wid >= 10)
    def _():
        pltpu.sync_copy(nw_hbm.at[pl.ds(row0, 3120)], nwc.at[pl.ds(0, 3120)])
        pltpu.sync_copy(ids_hbm.at[pl.ds(row0, 3120)], idc.at[pl.ds(0, 3120)])

    @pl.loop(0, nv)
    def _(i):
        sl = pl.ds(16 * i, 16)
        nw_v = nwc[sl]
        id_v = idc[sl]
        m_v = plsc.load_gather(mv, [id_v])
        d_v = plsc.load_gather(dv, [id_v])
        wc[sl] = jnp.exp(nw_v - m_v) * d_v

    @pl.when(wid < 10)
    def _():
        pltpu.sync_copy(wc.at[pl.ds(0, 3136)], w_hbm.at[pl.ds(row0, 3136)])

    @pl.when(wid >= 10)
    def _():
        pltpu.sync_copy(wc.at[pl.ds(0, 3120)], w_hbm.at[pl.ds(row0, 3120)])


@functools.partial(
    pl.kernel,
    out_type=[
        jax.ShapeDtypeStruct((N,), jnp.float32),
        jax.ShapeDtypeStruct((G, 256), jnp.float32),
        jax.ShapeDtypeStruct((G, 16), jnp.float32),
        jax.ShapeDtypeStruct((NW, 2, REC_W), jnp.float32),
        jax.ShapeDtypeStruct((NW, 16), jnp.int32),
    ],
    mesh=_mesh,
    compiler_params=_cparams,
    scratch_types=[
        pltpu.VMEM((2, 64, D), jnp.float32),
        pltpu.VMEM((2, 64), jnp.int32),
        pltpu.VMEM((3136,), jnp.float32),
        pltpu.VMEM((D,), jnp.float32),
        pltpu.VMEM((L,), jnp.float32),
        pltpu.SMEM((8,), jnp.float32),
        pltpu.VMEM((L,), jnp.float32),
        pltpu.VMEM((D,), jnp.float32),
        pltpu.VMEM((D,), jnp.float32),
        pltpu.VMEM((2, 256), jnp.float32),
        pltpu.VMEM((2, 16), jnp.float32),
        pltpu.VMEM((2, REC_W), jnp.float32),
        pltpu.VMEM((L,), jnp.int32),
        pltpu.VMEM((256,), jnp.float32),
        pltpu.VMEM((L,), jnp.float32),
        pltpu.SMEM((8,), jnp.int32),
        pltpu.SMEM((16,), jnp.int32),
        pltpu.SemaphoreType.DMA((2,)),
        pltpu.SemaphoreType.DMA((2,)),
        pltpu.SemaphoreType.DMA,
    ],
)
def _kernel_a(nf, ids, wb, *refs):
    _a_body(nf, ids, wb, *refs)


@functools.partial(
    pl.kernel,
    out_type=[
        jax.ShapeDtypeStruct((G, 256), jnp.float32),
        jax.ShapeDtypeStruct((G,), jnp.float32),
        jax.ShapeDtypeStruct((G,), jnp.float32),
    ],
    mesh=_mesh,
    compiler_params=_cparams,
    scratch_types=[
        pltpu.VMEM((NW, 16), jnp.int32),
        pltpu.VMEM((32, 256), jnp.float32),
        pltpu.VMEM((32, 16), jnp.float32),
        pltpu.VMEM((REC_W,), jnp.float32),
        pltpu.VMEM((D,), jnp.float32),
        pltpu.VMEM((D,), jnp.float32),
        pltpu.VMEM((L,), jnp.float32),
        pltpu.VMEM((32,), jnp.float32),
        pltpu.VMEM((32,), jnp.float32),
        pltpu.SMEM((8,), jnp.float32),
        pltpu.SMEM((8,), jnp.int32),
    ],
)
def _kernel_b(outa, msa, recs, rid, *refs):
    _b_body(outa, msa, recs, rid, *refs)


@functools.partial(
    pl.kernel,
    out_type=jax.ShapeDtypeStruct((N,), jnp.float32),
    mesh=_mesh,
    compiler_params=_cparams,
    scratch_types=[
        pltpu.VMEM((G,), jnp.float32),
        pltpu.VMEM((G,), jnp.float32),
        pltpu.VMEM((3136,), jnp.float32),
        pltpu.VMEM((3136,), jnp.int32),
        pltpu.VMEM((3136,), jnp.float32),
    ],
)
def _kernel_c(nw, ids, mf, df, *refs):
    _c_body(nw, ids, mf, df, *refs)


def kernel(nf, segment_ids, W, b):
    ids = segment_ids.astype(jnp.int32)
    wb = jnp.concatenate(
        [W.reshape(D), jnp.broadcast_to(b.reshape(1), (16,))]).astype(
            jnp.float32)
    nw, outa, msa, recs, rid = _kernel_a(nf, ids, wb)
    out, mf, df = _kernel_b(outa, msa, recs, rid)
    w1d = _kernel_c(nw, ids, mf, df)
    return out, w1d.reshape(N, 1)


# speedup vs baseline: 12.2458x; 1.1900x over previous
import dataclasses
import functools

import jax
import jax.numpy as jnp
from jax import lax
from jax.experimental import pallas as pl
from jax.experimental.pallas import tpu as pltpu
from jax.experimental.pallas import tpu_sc as plsc

N = 100000
G = 1024
D = 128
L = 16
NW = 32
REC_W = 272
NEG_INF = float("-inf")

_mesh = plsc.VectorSubcoreMesh(core_axis_name="c", subcore_axis_name="s")

_cparams = pltpu.CompilerParams()
if "needs_layout_passes" in pltpu.CompilerParams.__dataclass_fields__:
    _cparams = dataclasses.replace(_cparams, needs_layout_passes=False)


def _bs(x):
    return lax.broadcast_in_dim(jnp.float32(0) + x, (L,), ())


def _bs_i(x):
    return lax.broadcast_in_dim(jnp.int32(0) + x, (L,), ())


def _lane_iota():
    return lax.broadcasted_iota(jnp.int32, (L,), 0)


def _tile_id():
    return lax.axis_index("s") * 2 + lax.axis_index("c")


def _a_body(nf_hbm, ids_hbm, wb_hbm, nw_hbm, outa_hbm, msa_hbm, recs_hbm,
            rid_hbm, buf, idbuf, nwbuf, w_v, b_v, st_f, s_v, acc, fmx,
            stage, ms_stage, rec_v, rid_v, empty_row, nwv_s, st_i, sm_ids,
            in_sem, fl_sem, init_sem):
    wid = _tile_id()
    row0 = 3120 * wid + 16 * jnp.minimum(wid, 10)
    nc4 = jnp.where(wid < 10, 49, 48)

    pltpu.sync_copy(wb_hbm.at[pl.ds(0, 128)], w_v)
    pltpu.sync_copy(wb_hbm.at[pl.ds(128, 16)], b_v)
    b_s = b_v[...][0]

    pltpu.sync_copy(ids_hbm.at[pl.ds(row0, 16)], idbuf.at[0, pl.ds(0, 16)])
    last_row0 = row0 + jnp.where(wid < 10, 3136 - 16, 3120 - 16)
    pltpu.sync_copy(ids_hbm.at[pl.ds(last_row0, 16)],
                    idbuf.at[0, pl.ds(16, 16)])
    first_seg = idbuf[0, pl.ds(0, 16)][0]
    last_seg = idbuf[0, pl.ds(16, 16)][15]
    st_i[0] = first_seg
    st_i[1] = 0
    st_i[2] = first_seg
    st_f[0] = jnp.float32(NEG_INF)

    lane = _lane_iota()
    zeros = jnp.zeros((L,), jnp.float32)
    ninf = jnp.full((L,), NEG_INF, jnp.float32)

    for j in range(8):
        acc[pl.ds(16 * j, 16)] = zeros
        fmx[pl.ds(16 * j, 16)] = ninf
        empty_row[pl.ds(16 * j, 16)] = zeros
        empty_row[pl.ds(128 + 16 * j, 16)] = ninf
        for sl in range(2):
            rec_v[sl, pl.ds(16 * j, 16)] = zeros
            rec_v[sl, pl.ds(128 + 16 * j, 16)] = ninf
    s_v[...] = zeros
    ms_neutral = jnp.where(lane == 0, ninf, zeros)
    rec_v[0, pl.ds(256, 16)] = ms_neutral
    rec_v[1, pl.ds(256, 16)] = ms_neutral

    @pl.loop(first_seg + 1, last_seg)
    def _(g):
        pltpu.async_copy(empty_row, outa_hbm.at[g], init_sem)

    @pl.loop(first_seg + 1, last_seg)
    def _(g):
        pltpu.make_async_copy(empty_row, outa_hbm.at[g], init_sem).wait()

    def build_msvec():
        m = st_f[0]
        s_sc = jnp.max(s_v[...])
        return jnp.where(lane == 0, _bs(m),
                         jnp.where(lane == 1, _bs(s_sc), zeros))

    def write_record(islot):
        for j in range(8):
            rec_v[islot, pl.ds(16 * j, 16)] = acc[pl.ds(16 * j, 16)]
            rec_v[islot, pl.ds(128 + 16 * j, 16)] = fmx[pl.ds(16 * j, 16)]
        rec_v[islot, pl.ds(256, 16)] = build_msvec()

    def flush(cur):
        is_first = cur == st_i[2]

        @pl.when(is_first)
        def _():
            write_record(0)

        @pl.when(jnp.logical_not(is_first))
        def _():
            f = st_i[1]
            slot = lax.rem(f, 2)

            @pl.when(f >= 2)
            def _():
                pltpu.make_async_copy(stage.at[slot], outa_hbm.at[0],
                                      fl_sem.at[slot]).wait()
                pltpu.make_async_copy(ms_stage.at[slot], msa_hbm.at[0],
                                      fl_sem.at[slot]).wait()

            inv = 1.0 / s_v[...]
            for j in range(8):
                stage[slot, pl.ds(16 * j, 16)] = acc[pl.ds(16 * j, 16)] * inv
                stage[slot, pl.ds(128 + 16 * j, 16)] = fmx[pl.ds(16 * j, 16)]
            ms_stage[slot, pl.ds(0, 16)] = build_msvec()
            pltpu.async_copy(stage.at[slot], outa_hbm.at[cur],
                             fl_sem.at[slot])
            pltpu.async_copy(ms_stage.at[slot], msa_hbm.at[cur],
                             fl_sem.at[slot])
            st_i[1] = f + 1

    def reset_state(sid):
        for j in range(8):
            acc[pl.ds(16 * j, 16)] = zeros
            fmx[pl.ds(16 * j, 16)] = ninf
        s_v[...] = zeros
        st_f[0] = jnp.float32(NEG_INF)
        st_i[0] = sid

    def row_score(slot, rr, wregs):
        vs = [buf[slot, rr, pl.ds(16 * j, 16)] for j in range(8)]
        p = [vs[j] * wregs[j] for j in range(8)]
        q = [p[0] + p[1], p[2] + p[3], p[4] + p[5], p[6] + p[7]]
        a = (q[0] + q[1]) + (q[2] + q[3])
        return jnp.sum(a) + b_s, vs

    def process_group(slot, goff, vec_idx):
        wregs = [w_v[pl.ds(16 * j, 16)] for j in range(8)]
        gv = idbuf[slot, pl.ds(goff, 16)]
        g0s = gv[0]
        cur0 = st_i[0]
        allsame = jnp.min((gv == _bs_i(g0s)).astype(jnp.int32)) == 1
        single = jnp.logical_and(allsame, g0s == cur0)

        @pl.when(single)
        def _():
            nwv = zeros
            svg = zeros
            accg = [zeros] * 8
            fmxg = [jnp.full((L,), NEG_INF, jnp.float32)] * 8
            gref = jnp.float32(0.0)
            for r in range(16):
                nw_s, vs = row_score(slot, goff + r, wregs)
                if r == 0:
                    gref = nw_s
                nwv = jnp.where(lane == r, _bs(nw_s), nwv)
                e0 = jnp.exp(_bs(nw_s - gref))
                svg = svg + e0
                for j in range(8):
                    accg[j] = accg[j] + e0 * vs[j]
                    fmxg[j] = jnp.maximum(fmxg[j], vs[j])
            nwbuf[pl.ds(16 * vec_idx, 16)] = nwv
            gm = jnp.max(nwv)
            m_old = st_f[0]
            new_m = jnp.maximum(m_old, gm)
            sc_v = jnp.exp(_bs(m_old - new_m))
            gsc = jnp.exp(_bs(gref - new_m))
            s_v[...] = s_v[...] * sc_v + svg * gsc
            st_f[0] = new_m
            for j in range(8):
                sl = pl.ds(16 * j, 16)
                acc[sl] = acc[sl] * sc_v + accg[j] * gsc
                fmx[sl] = jnp.maximum(fmx[sl], fmxg[j])

        @pl.when(jnp.logical_not(single))
        def _():
            for r in range(16):
                sm_ids[r] = gv[r]
            nwv_s[...] = zeros

            @pl.loop(0, 16)
            def _(r):
                rr = goff + r
                sid = sm_ids[r]
                cur = st_i[0]

                @pl.when(sid != cur)
                def _():
                    flush(cur)
                    reset_state(sid)

                nw_s, vs = row_score(slot, rr, wregs)
                nwv_s[...] = jnp.where(lane == r, _bs(nw_s), nwv_s[...])

                m_old = st_f[0]
                new_m = jnp.maximum(m_old, nw_s)
                sc_v = jnp.exp(_bs(m_old - new_m))
                e_v = jnp.exp(_bs(nw_s - new_m))
                s_v[...] = s_v[...] * sc_v + e_v
                st_f[0] = new_m
                for j in range(8):
                    sl = pl.ds(16 * j, 16)
                    acc[sl] = acc[sl] * sc_v + e_v * vs[j]
                    fmx[sl] = jnp.maximum(fmx[sl], vs[j])

            nwbuf[pl.ds(16 * vec_idx, 16)] = nwv_s[...]

    def fire_chunk(ci, slot):
        pltpu.async_copy(nf_hbm.at[pl.ds(row0 + 64 * ci, 64), :],
                         buf.at[slot], in_sem.at[slot])
        pltpu.async_copy(ids_hbm.at[pl.ds(row0 + 64 * ci, 64)],
                         idbuf.at[slot], in_sem.at[slot])

    def wait_chunk(slot):
        pltpu.make_async_copy(nf_hbm.at[pl.ds(0, 64), :], buf.at[slot],
                              in_sem.at[slot]).wait()
        pltpu.make_async_copy(ids_hbm.at[pl.ds(0, 64)], idbuf.at[slot],
                              in_sem.at[slot]).wait()

    fire_chunk(0, 0)
    fire_chunk(1, 1)

    @pl.loop(0, nc4)
    def _(ci):
        slot = lax.rem(ci, 2)
        wait_chunk(slot)

        @pl.loop(0, 4)
        def _(g):
            process_group(slot, 16 * g, 4 * ci + g)

        @pl.when(ci + 2 < nc4)
        def _():
            fire_chunk(ci + 2, slot)

    @pl.when(wid >= 10)
    def _():
        pltpu.sync_copy(nf_hbm.at[pl.ds(row0 + 3072, 48), :],
                        buf.at[0, pl.ds(0, 48), :])
        pltpu.sync_copy(ids_hbm.at[pl.ds(row0 + 3072, 48)],
                        idbuf.at[0, pl.ds(0, 48)])

        @pl.loop(0, 3)
        def _(t):
            process_group(0, 16 * t, 192 + t)

    cur = st_i[0]
    is_first = cur == st_i[2]
    islot = jnp.where(is_first, 0, 1)
    write_record(islot)
    rid_v[...] = jnp.where(lane == 0, _bs_i(st_i[2]),
                           jnp.where(lane == 1, _bs_i(cur),
                                     jnp.zeros((L,), jnp.int32)))

    f = st_i[1]

    @pl.when(f >= 1)
    def _():
        slot = lax.rem(f - 1, 2)
        pltpu.make_async_copy(stage.at[slot], outa_hbm.at[0],
                              fl_sem.at[slot]).wait()
        pltpu.make_async_copy(ms_stage.at[slot], msa_hbm.at[0],
                              fl_sem.at[slot]).wait()

    @pl.when(f >= 2)
    def _():
        slot = lax.rem(f, 2)
        pltpu.make_async_copy(stage.at[slot], outa_hbm.at[0],
                              fl_sem.at[slot]).wait()
        pltpu.make_async_copy(ms_stage.at[slot], msa_hbm.at[0],
                              fl_sem.at[slot]).wait()

    pltpu.sync_copy(rec_v, recs_hbm.at[wid])
    pltpu.sync_copy(rid_v, rid_hbm.at[wid])

    @pl.when(wid < 10)
    def _():
        pltpu.sync_copy(nwbuf.at[pl.ds(0, 3136)],
                        nw_hbm.at[pl.ds(row0, 3136)])

    @pl.when(wid >= 10)
    def _():
        pltpu.sync_copy(nwbuf.at[pl.ds(0, 3120)],
                        nw_hbm.at[pl.ds(row0, 3120)])


def _b_body(outa_hbm, msa_hbm, recs_hbm, rid_hbm, out_hbm, mf_hbm, df_hbm,
            rid_v, obuf, msbuf, rbuf, tacc, tmx, s_v, mstage, dstage,
            st_f, st_b):
    t = _tile_id()
    g0 = 32 * t
    lane = _lane_iota()
    zeros = jnp.zeros((L,), jnp.float32)
    ninf = jnp.full((L,), NEG_INF, jnp.float32)

    pltpu.sync_copy(rid_hbm, rid_v)
    pltpu.sync_copy(outa_hbm.at[pl.ds(g0, 32), :], obuf)
    pltpu.sync_copy(msa_hbm.at[pl.ds(g0, 32), :], msbuf)

    def set_lane(ref, pos, valv):
        base = 16 * lax.div(pos, 16)
        lpos = lax.rem(pos, 16)
        v = ref[pl.ds(base, 16)]
        ref[pl.ds(base, 16)] = jnp.where(lane == lpos, valv, v)

    @pl.loop(0, 32)
    def _(gl):
        g = g0 + gl

        st_b[0] = 0
        st_b[1] = 0

        @pl.loop(0, NW)
        def _(w):
            rv = rid_v[w, pl.ds(0, 16)]
            fw = rv[0]
            lw = rv[1]
            bh = jnp.logical_or(fw == g, lw == g).astype(jnp.int32)
            ih = jnp.logical_and(fw < g, g < lw).astype(jnp.int32)
            st_b[0] = st_b[0] | bh
            st_b[1] = st_b[1] | ih

        bhit = st_b[0] != 0
        ihit = st_b[1] != 0

        @pl.when(bhit)
        def _():
            for j in range(8):
                tacc[pl.ds(16 * j, 16)] = zeros
                tmx[pl.ds(16 * j, 16)] = ninf
            s_v[...] = zeros
            st_f[0] = jnp.float32(NEG_INF)

            @pl.loop(0, NW)
            def _(w):
                rv = rid_v[w, pl.ds(0, 16)]

                def merge():
                    msv = rbuf[pl.ds(256, 16)]
                    m_j = msv[0]
                    sg_j = msv[1]
                    m_old = st_f[0]
                    new_m = jnp.maximum(m_old, m_j)
                    sc_old = jnp.exp(_bs(m_old - new_m))
                    sc_j = jnp.exp(_bs(m_j - new_m))
                    s_v[...] = s_v[...] * sc_old + _bs(sg_j) * sc_j
                    st_f[0] = new_m
                    for j in range(8):
                        sli = pl.ds(16 * j, 16)
                        slo = pl.ds(128 + 16 * j, 16)
                        tacc[sli] = tacc[sli] * sc_old + rbuf[sli] * sc_j
                        tmx[sli] = jnp.maximum(tmx[sli], rbuf[slo])

                @pl.when(rv[0] == g)
                def _():
                    pltpu.sync_copy(recs_hbm.at[w, 0], rbuf)
                    merge()

                @pl.when(rv[1] == g)
                def _():
                    pltpu.sync_copy(recs_hbm.at[w, 1], rbuf)
                    merge()

            inv = 1.0 / s_v[...]
            for j in range(8):
                obuf[gl, pl.ds(16 * j, 16)] = tacc[pl.ds(16 * j, 16)] * inv
                obuf[gl, pl.ds(128 + 16 * j, 16)] = tmx[pl.ds(16 * j, 16)]
            set_lane(mstage, gl, _bs(st_f[0]))
            set_lane(dstage, gl, inv)

        @pl.when(jnp.logical_and(jnp.logical_not(bhit), ihit))
        def _():
            mrow = msbuf[gl, pl.ds(0, 16)]
            set_lane(mstage, gl, _bs(mrow[0]))
            set_lane(dstage, gl, 1.0 / _bs(mrow[1]))

        @pl.when(jnp.logical_and(jnp.logical_not(bhit),
                                 jnp.logical_not(ihit)))
        def _():
            for j in range(8):
                obuf[gl, pl.ds(16 * j, 16)] = zeros
                obuf[gl, pl.ds(128 + 16 * j, 16)] = ninf
            set_lane(mstage, gl, zeros)
            set_lane(dstage, gl, zeros)

    pltpu.sync_copy(obuf, out_hbm.at[pl.ds(g0, 32), :])
    pltpu.sync_copy(mstage, mf_hbm.at[pl.ds(g0, 32)])
    pltpu.sync_copy(dstage, df_hbm.at[pl.ds(g0, 32)])


def _c_body(nw_hbm, ids_hbm, mf_hbm, df_hbm, w_hbm, mv, dv, nwc, idc, wc):
    wid = _tile_id()
    row0 = 3120 * wid + 16 * jnp.minimum(wid, 10)
    nv = jnp.where(wid < 10, 196, 195)

    pltpu.sync_copy(mf_hbm, mv)
    pltpu.sync_copy(df_hbm, dv)

    @pl.when(wid < 10)
    def _():
        pltpu.sync_copy(nw_hbm.at[pl.ds(row0, 3136)], nwc.at[pl.ds(0, 3136)])
        pltpu.sync_copy(ids_hbm.at[pl.ds(row0, 3136)], idc.at[pl.ds(0, 3136)])

    @pl.when(wid >= 10)
    def _():
        pltpu.sync_copy(nw_hbm.at[pl.ds(row0, 3120)], nwc.at[pl.ds(0, 3120)])
        pltpu.sync_copy(ids_hbm.at[pl.ds(row0, 3120)], idc.at[pl.ds(0, 3120)])

    @pl.loop(0, nv)
    def _(i):
        sl = pl.ds(16 * i, 16)
        nw_v = nwc[sl]
        id_v = idc[sl]
        m_v = plsc.load_gather(mv, [id_v])
        d_v = plsc.load_gather(dv, [id_v])
        wc[sl] = jnp.exp(nw_v - m_v) * d_v

    @pl.when(wid < 10)
    def _():
        pltpu.sync_copy(wc.at[pl.ds(0, 3136)], w_hbm.at[pl.ds(row0, 3136)])

    @pl.when(wid >= 10)
    def _():
        pltpu.sync_copy(wc.at[pl.ds(0, 3120)], w_hbm.at[pl.ds(row0, 3120)])


@functools.partial(
    pl.kernel,
    out_type=[
        jax.ShapeDtypeStruct((N,), jnp.float32),
        jax.ShapeDtypeStruct((G, 256), jnp.float32),
        jax.ShapeDtypeStruct((G, 16), jnp.float32),
        jax.ShapeDtypeStruct((NW, 2, REC_W), jnp.float32),
        jax.ShapeDtypeStruct((NW, 16), jnp.int32),
    ],
    mesh=_mesh,
    compiler_params=_cparams,
    scratch_types=[
        pltpu.VMEM((2, 64, D), jnp.float32),
        pltpu.VMEM((2, 64), jnp.int32),
        pltpu.VMEM((3136,), jnp.float32),
        pltpu.VMEM((D,), jnp.float32),
        pltpu.VMEM((L,), jnp.float32),
        pltpu.SMEM((8,), jnp.float32),
        pltpu.VMEM((L,), jnp.float32),
        pltpu.VMEM((D,), jnp.float32),
        pltpu.VMEM((D,), jnp.float32),
        pltpu.VMEM((2, 256), jnp.float32),
        pltpu.VMEM((2, 16), jnp.float32),
        pltpu.VMEM((2, REC_W), jnp.float32),
        pltpu.VMEM((L,), jnp.int32),
        pltpu.VMEM((256,), jnp.float32),
        pltpu.VMEM((L,), jnp.float32),
        pltpu.SMEM((8,), jnp.int32),
        pltpu.SMEM((16,), jnp.int32),
        pltpu.SemaphoreType.DMA((2,)),
        pltpu.SemaphoreType.DMA((2,)),
        pltpu.SemaphoreType.DMA,
    ],
)
def _kernel_a(nf, ids, wb, *refs):
    _a_body(nf, ids, wb, *refs)


@functools.partial(
    pl.kernel,
    out_type=[
        jax.ShapeDtypeStruct((G, 256), jnp.float32),
        jax.ShapeDtypeStruct((G,), jnp.float32),
        jax.ShapeDtypeStruct((G,), jnp.float32),
    ],
    mesh=_mesh,
    compiler_params=_cparams,
    scratch_types=[
        pltpu.VMEM((NW, 16), jnp.int32),
        pltpu.VMEM((32, 256), jnp.float32),
        pltpu.VMEM((32, 16), jnp.float32),
        pltpu.VMEM((REC_W,), jnp.float32),
        pltpu.VMEM((D,), jnp.float32),
        pltpu.VMEM((D,), jnp.float32),
        pltpu.VMEM((L,), jnp.float32),
        pltpu.VMEM((32,), jnp.float32),
        pltpu.VMEM((32,), jnp.float32),
        pltpu.SMEM((8,), jnp.float32),
        pltpu.SMEM((8,), jnp.int32),
    ],
)
def _kernel_b(outa, msa, recs, rid, *refs):
    _b_body(outa, msa, recs, rid, *refs)


@functools.partial(
    pl.kernel,
    out_type=jax.ShapeDtypeStruct((N,), jnp.float32),
    mesh=_mesh,
    compiler_params=_cparams,
    scratch_types=[
        pltpu.VMEM((G,), jnp.float32),
        pltpu.VMEM((G,), jnp.float32),
        pltpu.VMEM((3136,), jnp.float32),
        pltpu.VMEM((3136,), jnp.int32),
        pltpu.VMEM((3136,), jnp.float32),
    ],
)
def _kernel_c(nw, ids, mf, df, *refs):
    _c_body(nw, ids, mf, df, *refs)


def kernel(nf, segment_ids, W, b):
    ids = segment_ids.astype(jnp.int32)
    wb = jnp.concatenate(
        [W.reshape(D), jnp.broadcast_to(b.reshape(1), (16,))]).astype(
            jnp.float32)
    nw, outa, msa, recs, rid = _kernel_a(nf, ids, wb)
    out, mf, df = _kernel_b(outa, msa, recs, rid)
    w1d = _kernel_c(nw, ids, mf, df)
    return out, w1d.reshape(N, 1)
